# baseline (speedup 1.0000x reference)
import functools

import jax
import jax.numpy as jnp
from jax import lax
from jax.experimental import pallas as pl
from jax.experimental.pallas import tpu as pltpu
from jax.experimental.pallas import tpu_sc as plsc

_N = 10000
_E = 320000
_H = 128
_NSURF = 200000

_NC = 2
_NS = 16
_NW = _NC * _NS
_EPW = _E // _NW
_CH = 64
_GCH = _E // _CH
_RPT = 1000

_HI = jax.lax.Precision.HIGHEST
_f32 = jnp.float32


def _dot(a, b):
    return jnp.dot(a, b, precision=_HI, preferred_element_type=_f32)


def _mesh():
    return plsc.VectorSubcoreMesh(
        core_axis_name="c", subcore_axis_name="s",
        num_cores=_NC, num_subcores=_NS)


_NMCH = _NSURF // 128
_MREM_OFF = _NMCH * 128


def _make_edge_pass(write_ef: bool, with_mask: bool = False, ch: int = _CH):
    gch = _E // ch
    nmax = gch // _NW + (1 if gch % _NW else 0)
    def body(hw, ef_in, src, dst, *refs):
        refs = list(refs)
        surf = refs.pop(0) if with_mask else None
        ef_out = refs.pop(0) if write_ef else None
        agg_out = refs.pop(0)
        cnt_out = refs.pop(0) if with_mask else None
        (idx_s0, idx_s1, idx_d0, idx_d1, isc0, isc1,
         ef0, ef1, gs0, gs1, gd0, gd1, agg) = refs[:13]
        refs = refs[13:]
        if with_mask:
            (midx0, midx1, mones, m64i, m64o, zsrc, cnt) = refs[:7]
            refs = refs[7:]
        (semi0, semi1, semc0, semc1, seml0, seml1,
         seme0, seme1, sems0, sems1) = refs[:10]
        refs = refs[10:]
        if with_mask:
            (semmi0, semmi1, semms0, semms1) = refs
            midx = (midx0, midx1)
            sem_midx = (semmi0, semmi1)
            sem_msc = (semms0, semms1)
        idx_s = (idx_s0, idx_s1)
        idx_d = (idx_d0, idx_d1)
        idx_sc = (isc0, isc1)
        ef = (ef0, ef1)
        gs = (gs0, gs1)
        gd = (gd0, gd1)
        sem_idx = (semi0, semi1)
        sem_isc = (semc0, semc1)
        sem_ld = (seml0, seml1)
        sem_efst = (seme0, seme1)
        sem_sc = (sems0, sems1)

        cid = lax.axis_index("c")
        sid = lax.axis_index("s")
        wid = cid * _NS + sid
        nch = jnp.where(wid < gch % _NW, gch // _NW + 1, gch // _NW)

        if with_mask:
            nm = jnp.where(wid < _NMCH % _NW, _NMCH // _NW + 1, _NMCH // _NW)

            def issue_midx(m, b):
                base = (wid + _NW * m) * 128
                pltpu.async_copy(surf.at[pl.ds(base, 128)], midx[b], sem_midx[b])

            def wait_midx(m, b):
                base = (wid + _NW * m) * 128
                pltpu.make_async_copy(
                    surf.at[pl.ds(base, 128)], midx[b], sem_midx[b]).wait()

            def issue_msc(m, b):
                pltpu.async_copy(mones, cnt.at[midx[b]], sem_msc[b], add=True)

            def wait_msc(m, b):
                pltpu.make_async_copy(mones, cnt.at[midx[b]], sem_msc[b]).wait()

        def issue_idx(c, b):
            base = (wid + _NW * c) * ch
            pltpu.async_copy(src.at[pl.ds(base, ch)], idx_s[b], sem_idx[b])
            pltpu.async_copy(dst.at[pl.ds(base, ch)], idx_d[b], sem_idx[b])

        def wait_idx(c, b):
            base = (wid + _NW * c) * ch
            pltpu.make_async_copy(src.at[pl.ds(base, ch)], idx_s[b], sem_idx[b]).wait()
            pltpu.make_async_copy(dst.at[pl.ds(base, ch)], idx_d[b], sem_idx[b]).wait()

        def issue_isc(c, b):
            base = (wid + _NW * c) * ch
            pltpu.async_copy(dst.at[pl.ds(base, ch)], idx_sc[b], sem_isc[b])

        def wait_isc(c, b):
            base = (wid + _NW * c) * ch
            pltpu.make_async_copy(dst.at[pl.ds(base, ch)], idx_sc[b], sem_isc[b]).wait()

        def issue_loads(c, b):
            base = (wid + _NW * c) * ch
            pltpu.async_copy(ef_in.at[pl.ds(base, ch)], ef[b], sem_ld[b])
            pltpu.async_copy(hw.at[idx_s[b]], gs[b], sem_ld[b])
            pltpu.async_copy(hw.at[idx_d[b]], gd[b], sem_ld[b])

        def wait_loads(c, b):
            base = (wid + _NW * c) * ch
            pltpu.make_async_copy(ef_in.at[pl.ds(base, ch)], ef[b], sem_ld[b]).wait()
            pltpu.make_async_copy(hw.at[idx_s[b]], gs[b], sem_ld[b]).wait()
            pltpu.make_async_copy(hw.at[idx_d[b]], gd[b], sem_ld[b]).wait()

        def issue_stores(c, b):
            base = (wid + _NW * c) * ch
            if write_ef:
                pltpu.async_copy(ef[b], ef_out.at[pl.ds(base, ch)], sem_efst[b])
            pltpu.async_copy(ef[b], agg.at[idx_sc[b]], sem_sc[b], add=True)

        def wait_stores(c, b):
            base = (wid + _NW * c) * ch
            if write_ef:
                pltpu.make_async_copy(ef[b], ef_out.at[pl.ds(base, ch)], sem_efst[b]).wait()
            pltpu.make_async_copy(ef[b], agg.at[idx_sc[b]], sem_sc[b]).wait()

        def compute(b):
            efb, gsb, gdb = ef[b], gs[b], gd[b]

            @pl.loop(0, ch)
            def _row(i):
                for j in range(8):
                    s = pl.ds(j * 16, 16)
                    efb[i, s] = jnp.maximum(efb[i, s] + gsb[i, s] + gdb[i, s], 0.0)

        zeros16 = jnp.zeros((16,), _f32)

        @pl.loop(0, ch)
        def _zfill(i):
            for j in range(8):
                ef0[i, pl.ds(j * 16, 16)] = zeros16

        @pl.when(sid < 10)
        def _zero():
            for k in range(_RPT // ch):
                pltpu.sync_copy(
                    ef0, agg.at[pl.ds(sid * _RPT + k * ch, ch)])
            rem = _RPT % ch
            if rem:
                pltpu.sync_copy(
                    ef0.at[pl.ds(0, rem)],
                    agg.at[pl.ds(sid * _RPT + (_RPT // ch) * ch, rem)])

        if with_mask:
            ones16 = jnp.ones((16,), _f32)
            for j in range(8):
                mones[pl.ds(j * 16, 16)] = ones16
            for j in range(4):
                m64o[pl.ds(j * 16, 16)] = ones16
            for j in range(13):
                zsrc[pl.ds(j * 16, 16)] = zeros16

            @pl.when(sid < 10)
            def _zcnt():
                for k in range(5):
                    pltpu.sync_copy(
                        zsrc.at[pl.ds(0, 200)],
                        cnt.at[pl.ds(sid * 1000 + k * 200, 200)])
        plsc.subcore_barrier()

        issue_idx(0, 0)
        issue_idx(1, 1)
        issue_isc(0, 0)
        issue_isc(1, 1)
        wait_idx(0, 0)
        issue_loads(0, 0)

        def step(c, b):
            @pl.when(c + 1 < nch)
            def _():
                wait_idx(c + 1, 1 - b)

            @pl.when(c >= 1)
            def _():
                wait_stores(c - 1, 1 - b)

            @pl.when(jnp.logical_and(c >= 1, c + 1 < nch))
            def _():
                issue_isc(c + 1, 1 - b)

            @pl.when(c + 1 < nch)
            def _():
                issue_loads(c + 1, 1 - b)

            wait_loads(c, b)
            compute(b)
            wait_isc(c, b)
            issue_stores(c, b)

            @pl.when(c + 2 < nch)
            def _():
                issue_idx(c + 2, b)

            if with_mask:
                @pl.when(c < nm)
                def _():
                    @pl.when(c >= 2)
                    def _():
                        wait_msc(c - 2, b)
                    issue_midx(c, b)

                @pl.when(jnp.logical_and(c >= 1, c - 1 < nm))
                def _():
                    wait_midx(c - 1, 1 - b)
                    issue_msc(c - 1, 1 - b)


        @pl.loop(0, nmax + 1, step=2)
        def _pair(c):
            @pl.when(c < nch)
            def _():
                step(c, 0)

            @pl.when(c + 1 < nch)
            def _():
                step(c + 1, 1)

        @pl.when(nch % 2 == 0)
        def _drain_even():
            wait_stores(nch - 1, 1)

        @pl.when(nch % 2 == 1)
        def _drain_odd():
            wait_stores(nch - 1, 0)

        if with_mask:
            @pl.when(nm % 2 == 0)
            def _mdrain_even():
                wait_msc(nm - 2, 0)
                wait_msc(nm - 1, 1)

            @pl.when(nm % 2 == 1)
            def _mdrain_odd():
                wait_msc(nm - 2, 1)
                wait_msc(nm - 1, 0)

            @pl.when(wid == 0)
            def _mrem():
                pltpu.sync_copy(surf.at[pl.ds(_MREM_OFF, 64)], m64i)
                pltpu.sync_copy(m64o, cnt.at[m64i], add=True)
        plsc.subcore_barrier()

        @pl.when(sid < 10)
        def _flush():
            pltpu.sync_copy(agg.at[pl.ds(sid * _RPT, _RPT)],
                            agg_out.at[cid, pl.ds(sid * _RPT, _RPT)])

        if with_mask:
            @pl.when(sid == 15)
            def _mflush():
                pltpu.sync_copy(cnt, cnt_out.at[cid])

    outs = [jax.ShapeDtypeStruct((_NC, _N, _H), _f32)]
    if write_ef:
        outs = [jax.ShapeDtypeStruct((_E, _H), _f32)] + outs
    if with_mask:
        outs = outs + [jax.ShapeDtypeStruct((_NC, _N), _f32)]
    scratch = [
        pltpu.VMEM((ch,), jnp.int32),
        pltpu.VMEM((ch,), jnp.int32),
        pltpu.VMEM((ch,), jnp.int32),
        pltpu.VMEM((ch,), jnp.int32),
        pltpu.VMEM((ch,), jnp.int32),
        pltpu.VMEM((ch,), jnp.int32),
        pltpu.VMEM((ch, _H), _f32),
        pltpu.VMEM((ch, _H), _f32),
        pltpu.VMEM((ch, _H), _f32),
        pltpu.VMEM((ch, _H), _f32),
        pltpu.VMEM((ch, _H), _f32),
        pltpu.VMEM((ch, _H), _f32),
        pltpu.VMEM_SHARED((_N, _H), _f32),
    ]
    if with_mask:
        scratch += [
            pltpu.VMEM((128,), jnp.int32),
            pltpu.VMEM((128,), jnp.int32),
            pltpu.VMEM((128,), _f32),
            pltpu.VMEM((64,), jnp.int32),
            pltpu.VMEM((64,), _f32),
            pltpu.VMEM((208,), _f32),
            pltpu.VMEM_SHARED((_N,), _f32),
        ]
    scratch += [pltpu.SemaphoreType.DMA] * (14 if with_mask else 10)
    return pl.kernel(
        body,
        out_type=tuple(outs),
        mesh=_mesh(),
        scratch_types=scratch,
    )


def _mask_body(surf, cnt_out, idx128, idx64, ones128, ones64, zb, cnt):
    cid = lax.axis_index("c")
    sid = lax.axis_index("s")
    wid = cid * _NS + sid
    ones16 = jnp.ones((16,), _f32)
    zeros16 = jnp.zeros((16,), _f32)
    for j in range(8):
        ones128[pl.ds(j * 16, 16)] = ones16
    for j in range(4):
        ones64[pl.ds(j * 16, 16)] = ones16

    @pl.loop(0, 64)
    def _zf(i):
        zb[pl.ds(i * 16, 16)] = zeros16

    @pl.when(sid < 10)
    def _zero():
        pltpu.sync_copy(zb.at[pl.ds(0, 1000)], cnt.at[pl.ds(sid * 1000, 1000)])
    plsc.subcore_barrier()

    nfull = _NSURF // 128

    @pl.loop(wid, nfull, step=_NW)
    def _chunk(c):
        pltpu.sync_copy(surf.at[pl.ds(c * 128, 128)], idx128)
        pltpu.sync_copy(ones128, cnt.at[idx128], add=True)

    @pl.when(wid == 0)
    def _rem():
        pltpu.sync_copy(surf.at[pl.ds(nfull * 128, 64)], idx64)
        pltpu.sync_copy(ones64, cnt.at[idx64], add=True)
    plsc.subcore_barrier()

    @pl.when(sid == 0)
    def _out():
        pltpu.sync_copy(cnt, cnt_out.at[cid])


def _make_mask_kernel():
    return pl.kernel(
        _mask_body,
        out_type=jax.ShapeDtypeStruct((_NC, _N), _f32),
        mesh=_mesh(),
        scratch_types=[
            pltpu.VMEM((128,), jnp.int32),
            pltpu.VMEM((64,), jnp.int32),
            pltpu.VMEM((128,), _f32),
            pltpu.VMEM((64,), _f32),
            pltpu.VMEM((1024,), _f32),
            pltpu.VMEM_SHARED((_N,), _f32),
        ],
    )


def _enc_body(ea, W_ee, b_ee, x_t, W_enc, b_enc, We0, ef_out, h_out, hw_out):
    i = pl.program_id(0)
    ef_out[...] = jnp.maximum(_dot(ea[...], W_ee[...]) + b_ee[...], 0.0)

    @pl.when(i == 0)
    def _node():
        h = jnp.maximum(_dot(x_t[...], W_enc[...]) + b_enc[...], 0.0)
        h_out[...] = h
        hw_out[...] = _dot(h, We0[...])


def _make_layer_body(with_next: bool):
    def body(agg, h_in, Wn_l, *refs):
        if with_next:
            We_next, h_out, hw_out = refs
        else:
            (h_out,) = refs
        s = agg[0] + agg[1]
        h = h_in[...] + jnp.maximum(_dot(s, Wn_l[...]), 0.0)
        h_out[...] = h
        if with_next:
            hw_out[...] = _dot(h, We_next[...])
    return body


def _decode_body(agg, h_in, Wn_l, a_t, v_t, u_t, yv_t, yu_t, dtc, cnt2, xf,
                 W_ht, b_ht, b_hs, Wx_v, Wx_u, wdt_v, wdt_u,
                 bd_v, bd_u, Wa_v, Wa_u,
                 yv_o, yu_o, loss_o):
    i = pl.program_id(0)
    s = agg[0] + agg[1]
    h = h_in[...] + jnp.maximum(_dot(s, Wn_l[...]), 0.0)
    force = _dot(h, W_ht[...]) + b_ht[...]
    cnt = jnp.sum(cnt2[...], axis=1, keepdims=True)
    mask = (cnt > 0.0).astype(_f32)
    a_pred = force + mask * b_hs[...]
    dt = dtc[...]
    v1 = v_t[...] + a_pred * dt
    u1 = u_t[...] + v_t[...] * dt + 0.5 * a_pred * dt * dt
    da = a_pred - a_t[...]
    rate_v = _dot(xf[...], Wx_v[...]) + dt * wdt_v[...] + bd_v[...] + _dot(da, Wa_v[...])
    rate_u = _dot(xf[...], Wx_u[...]) + dt * wdt_u[...] + bd_u[...] + _dot(da, Wa_u[...])
    yv = v1 + rate_v * dt
    yu = u1 + rate_u * dt
    yv_o[...] = yv
    yu_o[...] = yu
    part = (jnp.sum(da * da) / (3.0 * _N)
            + (jnp.sum((yv - yv_t[...]) ** 2)
               + jnp.sum((yu - yu_t[...]) ** 2)) / (6.0 * _N))

    @pl.when(i == 0)
    def _init():
        loss_o[...] = jnp.zeros((1, 1), _f32)
    loss_o[...] += jnp.reshape(part, (1, 1))


def kernel(x, x_initial, node_mass, edge_attr, delta_t, y,
           edge_index, edge_surf_index,
           W_enc, b_enc, W_ee, b_ee, We, Wn,
           W_ht, b_ht, W_hs, b_hs, W_dec, b_dec):
    X_t = x[:, :, -1]
    a_t = X_t[:, 0:3]
    v_t = X_t[:, 3:6]
    u_t = X_t[:, 6:9]
    x_t = jnp.concatenate([u_t, v_t, x_initial, node_mass[:, None]], axis=-1)
    x_flat = x.reshape(_N, 27)
    src = edge_index[0]
    dst = edge_index[1]
    surf = edge_surf_index.reshape(-1)

    eb = 6400
    ef, h, hw = pl.pallas_call(
        _enc_body,
        grid=(_E // eb,),
        in_specs=[pl.BlockSpec((eb, 16), lambda i: (i, 0)),
                  pl.BlockSpec((16, _H), lambda i: (0, 0)),
                  pl.BlockSpec((1, _H), lambda i: (0, 0)),
                  pl.BlockSpec((_N, 10), lambda i: (0, 0)),
                  pl.BlockSpec((10, _H), lambda i: (0, 0)),
                  pl.BlockSpec((1, _H), lambda i: (0, 0)),
                  pl.BlockSpec((_H, _H), lambda i: (0, 0))],
        out_specs=(pl.BlockSpec((eb, _H), lambda i: (i, 0)),
                   pl.BlockSpec((_N, _H), lambda i: (0, 0)),
                   pl.BlockSpec((_N, _H), lambda i: (0, 0))),
        out_shape=(jax.ShapeDtypeStruct((_E, _H), _f32),
                   jax.ShapeDtypeStruct((_N, _H), _f32),
                   jax.ShapeDtypeStruct((_N, _H), _f32)),
    )(edge_attr, W_ee, b_ee.reshape(1, _H),
      x_t, W_enc, b_enc.reshape(1, _H), We[0])

    edge_rw_mask = _make_edge_pass(True, with_mask=True, ch=40)
    edge_rw = _make_edge_pass(True)
    edge_ro = _make_edge_pass(False)

    for l in range(2):
        if l == 0:
            ef, agg, cnt2 = edge_rw_mask(hw, ef, src, dst, surf)
        else:
            ef, agg = edge_rw(hw, ef, src, dst)
        h, hw = pl.pallas_call(
            _make_layer_body(True),
            out_shape=(jax.ShapeDtypeStruct((_N, _H), _f32),
                       jax.ShapeDtypeStruct((_N, _H), _f32)),
        )(agg, h, Wn[l], We[l + 1])
    (agg,) = edge_ro(hw, ef, src, dst)

    Wd_x = W_dec[:27]
    wd_dt = W_dec[27:28]
    Wd_a = W_dec[2:9:3]
    rb = 1000
    full = lambda r, c: pl.BlockSpec((r, c), lambda i: (0, 0))
    yv, yu, loss = pl.pallas_call(
        _decode_body,
        grid=(_N // rb,),
        in_specs=[pl.BlockSpec((2, rb, _H), lambda i: (0, i, 0)),
                  pl.BlockSpec((rb, _H), lambda i: (i, 0)),
                  pl.BlockSpec((_H, _H), lambda i: (0, 0)),
                  pl.BlockSpec((rb, 3), lambda i: (i, 0)),
                  pl.BlockSpec((rb, 3), lambda i: (i, 0)),
                  pl.BlockSpec((rb, 3), lambda i: (i, 0)),
                  pl.BlockSpec((rb, 3), lambda i: (i, 0)),
                  pl.BlockSpec((rb, 3), lambda i: (i, 0)),
                  pl.BlockSpec((rb, 1), lambda i: (i, 0)),
                  pl.BlockSpec((rb, 2), lambda i: (i, 0)),
                  pl.BlockSpec((rb, 27), lambda i: (i, 0)),
                  full(_H, 3), full(1, 3), full(1, 3),
                  full(27, 3), full(27, 3), full(1, 3), full(1, 3),
                  full(1, 3), full(1, 3), full(3, 3), full(3, 3)],
        out_specs=(pl.BlockSpec((rb, 3), lambda i: (i, 0)),
                   pl.BlockSpec((rb, 3), lambda i: (i, 0)),
                   pl.BlockSpec((1, 1), lambda i: (0, 0))),
        out_shape=(jax.ShapeDtypeStruct((_N, 3), _f32),
                   jax.ShapeDtypeStruct((_N, 3), _f32),
                   jax.ShapeDtypeStruct((1, 1), _f32)),
    )(agg, h, Wn[2], a_t, v_t, u_t, y[:, 3:6], y[:, 6:9],
      delta_t.reshape(_N, 1), cnt2.T, x_flat,
      W_ht, b_ht.reshape(1, 3), b_hs.reshape(1, 3),
      Wd_x[:, 0:3], Wd_x[:, 3:6], wd_dt[:, 0:3], wd_dt[:, 3:6],
      b_dec.reshape(1, 6)[:, 0:3], b_dec.reshape(1, 6)[:, 3:6],
      Wd_a[:, 0:3], Wd_a[:, 3:6])

    y_t = jnp.concatenate([yv, yu], axis=-1)
    return (y_t, loss[0, 0])


# speedup vs baseline: 1.0359x; 1.0359x over previous
import functools

import jax
import jax.numpy as jnp
from jax import lax
from jax.experimental import pallas as pl
from jax.experimental.pallas import tpu as pltpu
from jax.experimental.pallas import tpu_sc as plsc

_N = 10000
_E = 320000
_H = 128
_NSURF = 200000

_NC = 2
_NS = 16
_NW = _NC * _NS
_EPW = _E // _NW
_CH = 64
_GCH = _E // _CH
_RPT = 1000

_HI = jax.lax.Precision.HIGHEST
_f32 = jnp.float32


def _dot(a, b):
    return jnp.dot(a, b, precision=_HI, preferred_element_type=_f32)


def _mesh():
    return plsc.VectorSubcoreMesh(
        core_axis_name="c", subcore_axis_name="s",
        num_cores=_NC, num_subcores=_NS)


_NMCH = _NSURF // 128
_MREM_OFF = _NMCH * 128


def _make_edge_pass(write_ef: bool, with_mask: bool = False, ch: int = _CH):
    gch = _E // ch
    nmax = gch // _NW + (1 if gch % _NW else 0)
    def body(hw, ef_in, src, dst, *refs):
        refs = list(refs)
        surf = refs.pop(0) if with_mask else None
        ef_out = refs.pop(0) if write_ef else None
        agg_out = refs.pop(0)
        cnt_out = refs.pop(0) if with_mask else None
        (idx_s0, idx_s1, idx_d0, idx_d1, isc0, isc1,
         ef0, ef1, gs0, gs1, gd0, gd1, agg) = refs[:13]
        refs = refs[13:]
        if with_mask:
            (midx0, midx1, mones, m64i, m64o, zsrc, cnt) = refs[:7]
            refs = refs[7:]
        (semi0, semi1, semc0, semc1, seml0, seml1,
         seme0, seme1, sems0, sems1) = refs[:10]
        refs = refs[10:]
        if with_mask:
            (semmi0, semmi1, semms0, semms1) = refs
            midx = (midx0, midx1)
            sem_midx = (semmi0, semmi1)
            sem_msc = (semms0, semms1)
        idx_s = (idx_s0, idx_s1)
        idx_d = (idx_d0, idx_d1)
        idx_sc = (isc0, isc1)
        ef = (ef0, ef1)
        gs = (gs0, gs1)
        gd = (gd0, gd1)
        sem_idx = (semi0, semi1)
        sem_isc = (semc0, semc1)
        sem_ld = (seml0, seml1)
        sem_efst = (seme0, seme1)
        sem_sc = (sems0, sems1)

        cid = lax.axis_index("c")
        sid = lax.axis_index("s")
        wid = cid * _NS + sid
        nch = jnp.where(wid < gch % _NW, gch // _NW + 1, gch // _NW)

        if with_mask:
            nm = jnp.where(wid < _NMCH % _NW, _NMCH // _NW + 1, _NMCH // _NW)

            def issue_midx(m, b):
                base = (wid + _NW * m) * 128
                pltpu.async_copy(surf.at[pl.ds(base, 128)], midx[b], sem_midx[b])

            def wait_midx(m, b):
                base = (wid + _NW * m) * 128
                pltpu.make_async_copy(
                    surf.at[pl.ds(base, 128)], midx[b], sem_midx[b]).wait()

            def issue_msc(m, b):
                pltpu.async_copy(mones, cnt.at[midx[b]], sem_msc[b], add=True)

            def wait_msc(m, b):
                pltpu.make_async_copy(mones, cnt.at[midx[b]], sem_msc[b]).wait()

        def issue_idx(c, b):
            base = (wid + _NW * c) * ch
            pltpu.async_copy(src.at[pl.ds(base, ch)], idx_s[b], sem_idx[b])
            pltpu.async_copy(dst.at[pl.ds(base, ch)], idx_d[b], sem_idx[b])

        def wait_idx(c, b):
            base = (wid + _NW * c) * ch
            pltpu.make_async_copy(src.at[pl.ds(base, ch)], idx_s[b], sem_idx[b]).wait()
            pltpu.make_async_copy(dst.at[pl.ds(base, ch)], idx_d[b], sem_idx[b]).wait()

        def issue_isc(c, b):
            base = (wid + _NW * c) * ch
            pltpu.async_copy(dst.at[pl.ds(base, ch)], idx_sc[b], sem_isc[b])

        def wait_isc(c, b):
            base = (wid + _NW * c) * ch
            pltpu.make_async_copy(dst.at[pl.ds(base, ch)], idx_sc[b], sem_isc[b]).wait()

        def issue_loads(c, b):
            base = (wid + _NW * c) * ch
            pltpu.async_copy(ef_in.at[pl.ds(base, ch)], ef[b], sem_ld[b])
            pltpu.async_copy(hw.at[idx_s[b]], gs[b], sem_ld[b])
            pltpu.async_copy(hw.at[idx_d[b]], gd[b], sem_ld[b])

        def wait_loads(c, b):
            base = (wid + _NW * c) * ch
            pltpu.make_async_copy(ef_in.at[pl.ds(base, ch)], ef[b], sem_ld[b]).wait()
            pltpu.make_async_copy(hw.at[idx_s[b]], gs[b], sem_ld[b]).wait()
            pltpu.make_async_copy(hw.at[idx_d[b]], gd[b], sem_ld[b]).wait()

        def issue_stores(c, b):
            base = (wid + _NW * c) * ch
            if write_ef:
                pltpu.async_copy(ef[b], ef_out.at[pl.ds(base, ch)], sem_efst[b])
            pltpu.async_copy(ef[b], agg.at[idx_sc[b]], sem_sc[b], add=True)

        def wait_stores(c, b):
            base = (wid + _NW * c) * ch
            if write_ef:
                pltpu.make_async_copy(ef[b], ef_out.at[pl.ds(base, ch)], sem_efst[b]).wait()
            pltpu.make_async_copy(ef[b], agg.at[idx_sc[b]], sem_sc[b]).wait()

        def compute(b):
            efb, gsb, gdb = ef[b], gs[b], gd[b]

            @pl.loop(0, ch)
            def _row(i):
                for j in range(8):
                    s = pl.ds(j * 16, 16)
                    efb[i, s] = jnp.maximum(efb[i, s] + gsb[i, s] + gdb[i, s], 0.0)

        zeros16 = jnp.zeros((16,), _f32)

        @pl.loop(0, ch)
        def _zfill(i):
            for j in range(8):
                ef0[i, pl.ds(j * 16, 16)] = zeros16

        @pl.when(sid < 10)
        def _zero():
            for k in range(_RPT // ch):
                pltpu.sync_copy(
                    ef0, agg.at[pl.ds(sid * _RPT + k * ch, ch)])
            rem = _RPT % ch
            if rem:
                pltpu.sync_copy(
                    ef0.at[pl.ds(0, rem)],
                    agg.at[pl.ds(sid * _RPT + (_RPT // ch) * ch, rem)])

        if with_mask:
            ones16 = jnp.ones((16,), _f32)
            for j in range(8):
                mones[pl.ds(j * 16, 16)] = ones16
            for j in range(4):
                m64o[pl.ds(j * 16, 16)] = ones16
            for j in range(13):
                zsrc[pl.ds(j * 16, 16)] = zeros16

            @pl.when(sid < 10)
            def _zcnt():
                for k in range(5):
                    pltpu.sync_copy(
                        zsrc.at[pl.ds(0, 200)],
                        cnt.at[pl.ds(sid * 1000 + k * 200, 200)])
        plsc.subcore_barrier()

        issue_idx(0, 0)
        issue_idx(1, 1)
        issue_isc(0, 0)
        issue_isc(1, 1)
        wait_idx(0, 0)
        issue_loads(0, 0)

        def step(c, b):
            @pl.when(c + 1 < nch)
            def _():
                wait_idx(c + 1, 1 - b)

            @pl.when(c >= 1)
            def _():
                wait_stores(c - 1, 1 - b)

            @pl.when(jnp.logical_and(c >= 1, c + 1 < nch))
            def _():
                issue_isc(c + 1, 1 - b)

            @pl.when(c + 1 < nch)
            def _():
                issue_loads(c + 1, 1 - b)

            wait_loads(c, b)
            compute(b)
            wait_isc(c, b)
            issue_stores(c, b)

            @pl.when(c + 2 < nch)
            def _():
                issue_idx(c + 2, b)

            if with_mask:
                @pl.when(c < nm)
                def _():
                    @pl.when(c >= 2)
                    def _():
                        wait_msc(c - 2, b)
                    issue_midx(c, b)

                @pl.when(jnp.logical_and(c >= 1, c - 1 < nm))
                def _():
                    wait_midx(c - 1, 1 - b)
                    issue_msc(c - 1, 1 - b)


        @pl.loop(0, nmax + 1, step=2)
        def _pair(c):
            @pl.when(c < nch)
            def _():
                step(c, 0)

            @pl.when(c + 1 < nch)
            def _():
                step(c + 1, 1)

        @pl.when(nch % 2 == 0)
        def _drain_even():
            wait_stores(nch - 1, 1)

        @pl.when(nch % 2 == 1)
        def _drain_odd():
            wait_stores(nch - 1, 0)

        if with_mask:
            @pl.when(nm % 2 == 0)
            def _mdrain_even():
                wait_msc(nm - 2, 0)
                wait_msc(nm - 1, 1)

            @pl.when(nm % 2 == 1)
            def _mdrain_odd():
                wait_msc(nm - 2, 1)
                wait_msc(nm - 1, 0)

            @pl.when(wid == 0)
            def _mrem():
                pltpu.sync_copy(surf.at[pl.ds(_MREM_OFF, 64)], m64i)
                pltpu.sync_copy(m64o, cnt.at[m64i], add=True)
        plsc.subcore_barrier()

        @pl.when(sid < 10)
        def _flush():
            pltpu.sync_copy(agg.at[pl.ds(sid * _RPT, _RPT)],
                            agg_out.at[cid, pl.ds(sid * _RPT, _RPT)])

        if with_mask:
            @pl.when(sid == 15)
            def _mflush():
                pltpu.sync_copy(cnt, cnt_out.at[cid])

    outs = [jax.ShapeDtypeStruct((_NC, _N, _H), _f32)]
    if write_ef:
        outs = [jax.ShapeDtypeStruct((_E, _H), _f32)] + outs
    if with_mask:
        outs = outs + [jax.ShapeDtypeStruct((_NC, _N), _f32)]
    scratch = [
        pltpu.VMEM((ch,), jnp.int32),
        pltpu.VMEM((ch,), jnp.int32),
        pltpu.VMEM((ch,), jnp.int32),
        pltpu.VMEM((ch,), jnp.int32),
        pltpu.VMEM((ch,), jnp.int32),
        pltpu.VMEM((ch,), jnp.int32),
        pltpu.VMEM((ch, _H), _f32),
        pltpu.VMEM((ch, _H), _f32),
        pltpu.VMEM((ch, _H), _f32),
        pltpu.VMEM((ch, _H), _f32),
        pltpu.VMEM((ch, _H), _f32),
        pltpu.VMEM((ch, _H), _f32),
        pltpu.VMEM_SHARED((_N, _H), _f32),
    ]
    if with_mask:
        scratch += [
            pltpu.VMEM((128,), jnp.int32),
            pltpu.VMEM((128,), jnp.int32),
            pltpu.VMEM((128,), _f32),
            pltpu.VMEM((64,), jnp.int32),
            pltpu.VMEM((64,), _f32),
            pltpu.VMEM((208,), _f32),
            pltpu.VMEM_SHARED((_N,), _f32),
        ]
    scratch += [pltpu.SemaphoreType.DMA] * (14 if with_mask else 10)
    return pl.kernel(
        body,
        out_type=tuple(outs),
        mesh=_mesh(),
        scratch_types=scratch,
    )


def _mask_body(surf, cnt_out, idx128, idx64, ones128, ones64, zb, cnt):
    cid = lax.axis_index("c")
    sid = lax.axis_index("s")
    wid = cid * _NS + sid
    ones16 = jnp.ones((16,), _f32)
    zeros16 = jnp.zeros((16,), _f32)
    for j in range(8):
        ones128[pl.ds(j * 16, 16)] = ones16
    for j in range(4):
        ones64[pl.ds(j * 16, 16)] = ones16

    @pl.loop(0, 64)
    def _zf(i):
        zb[pl.ds(i * 16, 16)] = zeros16

    @pl.when(sid < 10)
    def _zero():
        pltpu.sync_copy(zb.at[pl.ds(0, 1000)], cnt.at[pl.ds(sid * 1000, 1000)])
    plsc.subcore_barrier()

    nfull = _NSURF // 128

    @pl.loop(wid, nfull, step=_NW)
    def _chunk(c):
        pltpu.sync_copy(surf.at[pl.ds(c * 128, 128)], idx128)
        pltpu.sync_copy(ones128, cnt.at[idx128], add=True)

    @pl.when(wid == 0)
    def _rem():
        pltpu.sync_copy(surf.at[pl.ds(nfull * 128, 64)], idx64)
        pltpu.sync_copy(ones64, cnt.at[idx64], add=True)
    plsc.subcore_barrier()

    @pl.when(sid == 0)
    def _out():
        pltpu.sync_copy(cnt, cnt_out.at[cid])


def _make_mask_kernel():
    return pl.kernel(
        _mask_body,
        out_type=jax.ShapeDtypeStruct((_NC, _N), _f32),
        mesh=_mesh(),
        scratch_types=[
            pltpu.VMEM((128,), jnp.int32),
            pltpu.VMEM((64,), jnp.int32),
            pltpu.VMEM((128,), _f32),
            pltpu.VMEM((64,), _f32),
            pltpu.VMEM((1024,), _f32),
            pltpu.VMEM_SHARED((_N,), _f32),
        ],
    )


def _enc_body(ea, W_ee, b_ee, x_t, W_enc, b_enc, We0, ef_out, h_out, hw_out):
    i = pl.program_id(0)
    ef_out[...] = jnp.maximum(_dot(ea[...], W_ee[...]) + b_ee[...], 0.0)

    @pl.when(i == 0)
    def _node():
        h = jnp.maximum(_dot(x_t[...], W_enc[...]) + b_enc[...], 0.0)
        h_out[...] = h
        hw_out[...] = _dot(h, We0[...])


def _make_layer_body(with_next: bool):
    def body(agg, h_in, Wn_l, *refs):
        if with_next:
            We_next, h_out, hw_out = refs
        else:
            (h_out,) = refs
        s = agg[0] + agg[1]
        h = h_in[...] + jnp.maximum(_dot(s, Wn_l[...]), 0.0)
        h_out[...] = h
        if with_next:
            hw_out[...] = _dot(h, We_next[...])
    return body


def _decode_body(agg, h_in, Wn_l, a_t, v_t, u_t, yv_t, yu_t, dtc, cnt2, xf,
                 W_ht, b_ht, b_hs, Wx_v, Wx_u, wdt_v, wdt_u,
                 bd_v, bd_u, Wa_v, Wa_u,
                 yv_o, yu_o, loss_o):
    i = pl.program_id(0)
    s = agg[0] + agg[1]
    h = h_in[...] + jnp.maximum(_dot(s, Wn_l[...]), 0.0)
    force = _dot(h, W_ht[...]) + b_ht[...]
    cnt = jnp.sum(cnt2[...], axis=1, keepdims=True)
    mask = (cnt > 0.0).astype(_f32)
    a_pred = force + mask * b_hs[...]
    dt = dtc[...]
    v1 = v_t[...] + a_pred * dt
    u1 = u_t[...] + v_t[...] * dt + 0.5 * a_pred * dt * dt
    da = a_pred - a_t[...]
    rate_v = _dot(xf[...], Wx_v[...]) + dt * wdt_v[...] + bd_v[...] + _dot(da, Wa_v[...])
    rate_u = _dot(xf[...], Wx_u[...]) + dt * wdt_u[...] + bd_u[...] + _dot(da, Wa_u[...])
    yv = v1 + rate_v * dt
    yu = u1 + rate_u * dt
    yv_o[...] = yv
    yu_o[...] = yu
    part = (jnp.sum(da * da) / (3.0 * _N)
            + (jnp.sum((yv - yv_t[...]) ** 2)
               + jnp.sum((yu - yu_t[...]) ** 2)) / (6.0 * _N))

    @pl.when(i == 0)
    def _init():
        loss_o[...] = jnp.zeros((1, 1), _f32)
    loss_o[...] += jnp.reshape(part, (1, 1))


def kernel(x, x_initial, node_mass, edge_attr, delta_t, y,
           edge_index, edge_surf_index,
           W_enc, b_enc, W_ee, b_ee, We, Wn,
           W_ht, b_ht, W_hs, b_hs, W_dec, b_dec):
    X_t = x[:, :, -1]
    a_t = X_t[:, 0:3]
    v_t = X_t[:, 3:6]
    u_t = X_t[:, 6:9]
    x_t = jnp.concatenate([u_t, v_t, x_initial, node_mass[:, None]], axis=-1)
    x_flat = x.reshape(_N, 27)
    src = edge_index[0]
    dst = edge_index[1]
    surf = edge_surf_index.reshape(-1)

    eb = 6400
    ef, h, hw = pl.pallas_call(
        _enc_body,
        grid=(_E // eb,),
        in_specs=[pl.BlockSpec((eb, 16), lambda i: (i, 0)),
                  pl.BlockSpec((16, _H), lambda i: (0, 0)),
                  pl.BlockSpec((1, _H), lambda i: (0, 0)),
                  pl.BlockSpec((_N, 10), lambda i: (0, 0)),
                  pl.BlockSpec((10, _H), lambda i: (0, 0)),
                  pl.BlockSpec((1, _H), lambda i: (0, 0)),
                  pl.BlockSpec((_H, _H), lambda i: (0, 0))],
        out_specs=(pl.BlockSpec((eb, _H), lambda i: (i, 0)),
                   pl.BlockSpec((_N, _H), lambda i: (0, 0)),
                   pl.BlockSpec((_N, _H), lambda i: (0, 0))),
        out_shape=(jax.ShapeDtypeStruct((_E, _H), _f32),
                   jax.ShapeDtypeStruct((_N, _H), _f32),
                   jax.ShapeDtypeStruct((_N, _H), _f32)),
    )(edge_attr, W_ee, b_ee.reshape(1, _H),
      x_t, W_enc, b_enc.reshape(1, _H), We[0])

    cnt2 = _make_mask_kernel()(surf)

    edge_rw = _make_edge_pass(True)
    edge_ro = _make_edge_pass(False)

    for l in range(2):
        ef, agg = edge_rw(hw, ef, src, dst)
        h, hw = pl.pallas_call(
            _make_layer_body(True),
            out_shape=(jax.ShapeDtypeStruct((_N, _H), _f32),
                       jax.ShapeDtypeStruct((_N, _H), _f32)),
        )(agg, h, Wn[l], We[l + 1])
    (agg,) = edge_ro(hw, ef, src, dst)

    Wd_x = W_dec[:27]
    wd_dt = W_dec[27:28]
    Wd_a = W_dec[2:9:3]
    rb = 1000
    full = lambda r, c: pl.BlockSpec((r, c), lambda i: (0, 0))
    yv, yu, loss = pl.pallas_call(
        _decode_body,
        grid=(_N // rb,),
        in_specs=[pl.BlockSpec((2, rb, _H), lambda i: (0, i, 0)),
                  pl.BlockSpec((rb, _H), lambda i: (i, 0)),
                  pl.BlockSpec((_H, _H), lambda i: (0, 0)),
                  pl.BlockSpec((rb, 3), lambda i: (i, 0)),
                  pl.BlockSpec((rb, 3), lambda i: (i, 0)),
                  pl.BlockSpec((rb, 3), lambda i: (i, 0)),
                  pl.BlockSpec((rb, 3), lambda i: (i, 0)),
                  pl.BlockSpec((rb, 3), lambda i: (i, 0)),
                  pl.BlockSpec((rb, 1), lambda i: (i, 0)),
                  pl.BlockSpec((rb, 2), lambda i: (i, 0)),
                  pl.BlockSpec((rb, 27), lambda i: (i, 0)),
                  full(_H, 3), full(1, 3), full(1, 3),
                  full(27, 3), full(27, 3), full(1, 3), full(1, 3),
                  full(1, 3), full(1, 3), full(3, 3), full(3, 3)],
        out_specs=(pl.BlockSpec((rb, 3), lambda i: (i, 0)),
                   pl.BlockSpec((rb, 3), lambda i: (i, 0)),
                   pl.BlockSpec((1, 1), lambda i: (0, 0))),
        out_shape=(jax.ShapeDtypeStruct((_N, 3), _f32),
                   jax.ShapeDtypeStruct((_N, 3), _f32),
                   jax.ShapeDtypeStruct((1, 1), _f32)),
    )(agg, h, Wn[2], a_t, v_t, u_t, y[:, 3:6], y[:, 6:9],
      delta_t.reshape(_N, 1), cnt2.T, x_flat,
      W_ht, b_ht.reshape(1, 3), b_hs.reshape(1, 3),
      Wd_x[:, 0:3], Wd_x[:, 3:6], wd_dt[:, 0:3], wd_dt[:, 3:6],
      b_dec.reshape(1, 6)[:, 0:3], b_dec.reshape(1, 6)[:, 3:6],
      Wd_a[:, 0:3], Wd_a[:, 3:6])

    y_t = jnp.concatenate([yv, yu], axis=-1)
    return (y_t, loss[0, 0])
